# baseline (speedup 1.0000x reference)
import jax
import jax.numpy as jnp
from jax.experimental import pallas as pl

_NUM_PATCHES = 128
_PATCH_POINTS = 32
_EMBED_DIM = 384


def _sqdist(src, dst):
    d = -2.0 * jnp.matmul(src, jnp.transpose(dst, (0, 2, 1)))
    d = d + jnp.sum(src ** 2, -1)[:, :, None]
    d = d + jnp.sum(dst ** 2, -1)[:, None, :]
    return d


def _fps(x, npoint):
    xyz = x[:, :, :3]
    b, n, _ = xyz.shape
    distance = jnp.full((b, n), 1e10, dtype=xyz.dtype)
    farthest = jnp.zeros((b,), dtype=jnp.int32)
    batch_idx = jnp.arange(b)
    cents = []
    for _ in range(npoint):
        cents.append(farthest)
        centroid = xyz[batch_idx, farthest][:, None, :]
        d = jnp.sum((xyz - centroid) ** 2, -1)
        distance = jnp.minimum(distance, d)
        farthest = jnp.argmax(distance, axis=-1).astype(jnp.int32)
    return jnp.stack(cents, axis=1)


def _bn(h, g, b, eps=1e-5):
    m = jnp.mean(h, axis=(0, 1), keepdims=True)
    v = jnp.var(h, axis=(0, 1), keepdims=True)
    return g * (h - m) / jnp.sqrt(v + eps) + b


def _copy_kernel(x_ref, o_ref):
    o_ref[...] = x_ref[...]


def kernel(x, W1, b1, g1, be1, W2, b2, g2, be2, W3, b3):
    b, n, c = x.shape
    xyz = x[:, :, :3]
    fps_idx = _fps(x, _NUM_PATCHES)
    bi = jnp.arange(b)[:, None]
    center_pos = xyz[bi, fps_idx]
    new_x = x[bi, fps_idx]
    sqr = _sqdist(new_x[:, :, :3], xyz[:, :, :3])
    _, idx = jax.lax.top_k(-sqr, _PATCH_POINTS)
    patches = x[jnp.arange(b)[:, None, None], idx]
    patches = patches.at[:, :, :, :3].add(-center_pos[:, :, None, :])
    pf = patches.reshape(b * _NUM_PATCHES, _PATCH_POINTS, c)
    h = pf @ W1.T + b1
    h = jax.nn.relu(_bn(h, g1, be1))
    h = h @ W2.T + b2
    h = jax.nn.relu(_bn(h, g2, be2))
    h = h @ W3.T + b3
    emb = jnp.max(h, axis=1).reshape(b, _NUM_PATCHES, _EMBED_DIM)
    emb = pl.pallas_call(
        _copy_kernel,
        out_shape=jax.ShapeDtypeStruct(emb.shape, emb.dtype),
    )(emb)
    return (emb, center_pos)


# speedup vs baseline: 5.0553x; 5.0553x over previous
import functools

import jax
import jax.numpy as jnp
from jax import lax
from jax.experimental import pallas as pl
from jax.experimental.pallas import tpu as pltpu
from jax.experimental.pallas import tpu_sc as plsc

B = 32
N = 2048
NP = 128
PP = 32
ROWS = B * NP * PP
BLK = 1024
GRID = ROWS // BLK


def _trunc_bf16(v):
    bits = plsc.bitcast(v, jnp.int32)
    rounded = bits + 0x7FFF + ((bits >> 16) & 1)
    return plsc.bitcast(rounded & jnp.int32(-65536), jnp.float32)


def _sc_body(xs_h, ys_h, zs_h,
             cx_h, cy_h, cz_h, px_h, py_h, pz_h,
             xs_v, ys_v, zs_v, dist_v, xb_v, yb_v, zb_v, sp_v,
             cx_v, cy_v, cz_v, px_v, py_v, pz_v):
    w = lax.axis_index("s") * 2 + lax.axis_index("c")
    pltpu.sync_copy(xs_h.at[w], xs_v)
    pltpu.sync_copy(ys_h.at[w], ys_v)
    pltpu.sync_copy(zs_h.at[w], zs_v)

    iota = lax.iota(jnp.int32, 16)
    lane0 = iota == 0
    inf16 = jnp.full((16,), 1e30, jnp.float32)
    zero16i = jnp.zeros((16,), jnp.int32)

    def init_chunk(c, carry):
        sl = pl.ds(c * 16, 16)
        dist_v[sl] = jnp.full((16,), 1e10, jnp.float32)
        xv = xs_v[sl]
        yv = ys_v[sl]
        zv = zs_v[sl]
        xb_v[sl] = _trunc_bf16(xv)
        yb_v[sl] = _trunc_bf16(yv)
        zb_v[sl] = _trunc_bf16(zv)
        sp_v[sl] = xv * xv + yv * yv + zv * zv
        return carry

    lax.fori_loop(0, N // 16, init_chunk, 0)

    def fps_step(t, far):
        fv = jnp.full((16,), far, jnp.int32)
        cxv = plsc.load_gather(xs_v, [fv])
        cyv = plsc.load_gather(ys_v, [fv])
        czv = plsc.load_gather(zs_v, [fv])
        tv = jnp.full((16,), t, jnp.int32)
        plsc.store_scatter(cx_v, [tv], cxv, mask=lane0)
        plsc.store_scatter(cy_v, [tv], cyv, mask=lane0)
        plsc.store_scatter(cz_v, [tv], czv, mask=lane0)

        def chunk(c, carry):
            m, mi = carry
            sl = pl.ds(c * 16, 16)
            dx = xs_v[sl] - cxv
            dy = ys_v[sl] - cyv
            dz = zs_v[sl] - czv
            d = dx * dx + dy * dy + dz * dz
            dn = jnp.minimum(dist_v[sl], d)
            dist_v[sl] = dn
            upd = dn > m
            m = jnp.where(upd, dn, m)
            mi = jnp.where(upd, c * 16 + iota, mi)
            return (m, mi)

        m, mi = lax.fori_loop(0, N // 16, chunk,
                              (jnp.full((16,), -1.0, jnp.float32), zero16i))
        mmax = jnp.max(m)
        cand = jnp.where(m == mmax, mi, N)
        return jnp.min(cand)

    lax.fori_loop(0, NP, fps_step, jnp.int32(0))

    def knn_q(q, carry):
        qv = jnp.full((16,), q, jnp.int32)
        qx = plsc.load_gather(cx_v, [qv])
        qy = plsc.load_gather(cy_v, [qv])
        qz = plsc.load_gather(cz_v, [qv])
        qxb = _trunc_bf16(qx)
        qyb = _trunc_bf16(qy)
        qzb = _trunc_bf16(qz)
        sq = qx * qx + qy * qy + qz * qz

        def chunk(c, st):
            r0k, r0v, r1k, r1v = st
            sl = pl.ds(c * 16, 16)
            prod = xb_v[sl] * qxb + yb_v[sl] * qyb
            prod = prod + zb_v[sl] * qzb
            d = -2.0 * prod + sq
            d = d + sp_v[sl]
            ck, cv = plsc.sort_key_val(d, c * 16 + iota)
            rck = lax.rev(ck, (0,))
            rcv = lax.rev(cv, (0,))
            sel = r1k <= rck
            lk = jnp.where(sel, r1k, rck)
            lv = jnp.where(sel, r1v, rcv)
            lk, lv = plsc.sort_key_val(lk, lv)
            rlk = lax.rev(lk, (0,))
            rlv = lax.rev(lv, (0,))
            sel2 = r0k <= rlk
            lok = jnp.where(sel2, r0k, rlk)
            lov = jnp.where(sel2, r0v, rlv)
            hik = jnp.where(sel2, rlk, r0k)
            hiv = jnp.where(sel2, rlv, r0v)
            r0k, r0v = plsc.sort_key_val(lok, lov)
            r1k, r1v = plsc.sort_key_val(hik, hiv)
            return (r0k, r0v, r1k, r1v)

        r0k, r0v, r1k, r1v = lax.fori_loop(
            0, N // 16, chunk, (inf16, zero16i, inf16, zero16i))

        base = q * PP
        px_v[pl.ds(base, 16)] = plsc.load_gather(xs_v, [r0v]) - qx
        px_v[pl.ds(base + 16, 16)] = plsc.load_gather(xs_v, [r1v]) - qx
        py_v[pl.ds(base, 16)] = plsc.load_gather(ys_v, [r0v]) - qy
        py_v[pl.ds(base + 16, 16)] = plsc.load_gather(ys_v, [r1v]) - qy
        pz_v[pl.ds(base, 16)] = plsc.load_gather(zs_v, [r0v]) - qz
        pz_v[pl.ds(base + 16, 16)] = plsc.load_gather(zs_v, [r1v]) - qz
        return carry

    lax.fori_loop(0, NP, knn_q, 0)

    pltpu.sync_copy(cx_v, cx_h.at[w])
    pltpu.sync_copy(cy_v, cy_h.at[w])
    pltpu.sync_copy(cz_v, cz_h.at[w])
    pltpu.sync_copy(px_v, px_h.at[w])
    pltpu.sync_copy(py_v, py_h.at[w])
    pltpu.sync_copy(pz_v, pz_h.at[w])


_sc_call = functools.partial(
    pl.kernel,
    mesh=plsc.VectorSubcoreMesh(core_axis_name="c", subcore_axis_name="s"),
    compiler_params=pltpu.CompilerParams(needs_layout_passes=False),
    out_type=[
        jax.ShapeDtypeStruct((B, NP), jnp.float32),
        jax.ShapeDtypeStruct((B, NP), jnp.float32),
        jax.ShapeDtypeStruct((B, NP), jnp.float32),
        jax.ShapeDtypeStruct((B, NP * PP), jnp.float32),
        jax.ShapeDtypeStruct((B, NP * PP), jnp.float32),
        jax.ShapeDtypeStruct((B, NP * PP), jnp.float32),
    ],
    scratch_types=[
        pltpu.VMEM((N,), jnp.float32),
        pltpu.VMEM((N,), jnp.float32),
        pltpu.VMEM((N,), jnp.float32),
        pltpu.VMEM((N,), jnp.float32),
        pltpu.VMEM((N,), jnp.float32),
        pltpu.VMEM((N,), jnp.float32),
        pltpu.VMEM((N,), jnp.float32),
        pltpu.VMEM((N,), jnp.float32),
        pltpu.VMEM((NP,), jnp.float32),
        pltpu.VMEM((NP,), jnp.float32),
        pltpu.VMEM((NP,), jnp.float32),
        pltpu.VMEM((NP * PP,), jnp.float32),
        pltpu.VMEM((NP * PP,), jnp.float32),
        pltpu.VMEM((NP * PP,), jnp.float32),
    ],
)(_sc_body)


def _dotf(a, b):
    return lax.dot_general(a, b, (((1,), (0,)), ((), ())),
                           preferred_element_type=jnp.float32)


def _mom_body(p_ref, m_ref):
    @pl.when(pl.program_id(0) == 0)
    def _():
        m_ref[...] = jnp.zeros_like(m_ref)

    p = p_ref[...]
    m_ref[...] += lax.dot_general(p, p, (((0,), (0,)), ((), ())),
                                  preferred_element_type=jnp.float32)


def _a1_of(p_ref, w1_ref, m1_ref, sc1_ref, be1_ref):
    h1 = _dotf(p_ref[...], w1_ref[...])
    return jnp.maximum(
        (h1 - m1_ref[0:1, :]) * sc1_ref[0:1, :] + be1_ref[0:1, :], 0.0)


def _stat1_body(p_ref, w1_ref, m1_ref, sc1_ref, be1_ref, g_ref, s_ref):
    @pl.when(pl.program_id(0) == 0)
    def _():
        g_ref[...] = jnp.zeros_like(g_ref)
        s_ref[...] = jnp.zeros_like(s_ref)

    a1 = _a1_of(p_ref, w1_ref, m1_ref, sc1_ref, be1_ref).astype(jnp.bfloat16)
    g_ref[...] += lax.dot_general(a1, a1, (((0,), (0,)), ((), ())),
                                  preferred_element_type=jnp.float32)
    s_ref[...] += jnp.broadcast_to(
        jnp.sum(a1.astype(jnp.float32), axis=0, keepdims=True), s_ref.shape)


def _mlp_body(p_ref, w1_ref, m1_ref, sc1_ref, be1_ref,
              w2_ref, cb2_ref, sc2_ref, be2_ref, w3_ref, b3_ref, o_ref):
    a1 = _a1_of(p_ref, w1_ref, m1_ref, sc1_ref, be1_ref).astype(jnp.bfloat16)
    h2 = _dotf(a1, w2_ref[...])
    a2 = jnp.maximum(
        (h2 + cb2_ref[0:1, :]) * sc2_ref[0:1, :] + be2_ref[0:1, :], 0.0)
    h3 = _dotf(a2.astype(jnp.bfloat16), w3_ref[...]) + b3_ref[0:1, :]
    o_ref[...] = jnp.max(h3.reshape(BLK // PP, PP, h3.shape[-1]), axis=1)


def _row8(v, n):
    return jnp.broadcast_to(v[None, :], (8, n)).astype(jnp.float32)


def _moments(p_aug):
    return pl.pallas_call(
        _mom_body,
        grid=(GRID,),
        in_specs=[pl.BlockSpec((BLK, 8), lambda i: (i, 0))],
        out_specs=pl.BlockSpec((8, 8), lambda i: (0, 0)),
        out_shape=jax.ShapeDtypeStruct((8, 8), jnp.float32),
    )(p_aug)


def _stats1(p_aug, w1b, m1r, sc1r, be1r):
    return pl.pallas_call(
        _stat1_body,
        grid=(GRID,),
        in_specs=[
            pl.BlockSpec((BLK, 8), lambda i: (i, 0)),
            pl.BlockSpec((8, 64), lambda i: (0, 0)),
            pl.BlockSpec((8, 64), lambda i: (0, 0)),
            pl.BlockSpec((8, 64), lambda i: (0, 0)),
            pl.BlockSpec((8, 64), lambda i: (0, 0)),
        ],
        out_specs=[
            pl.BlockSpec((64, 64), lambda i: (0, 0)),
            pl.BlockSpec((8, 64), lambda i: (0, 0)),
        ],
        out_shape=[
            jax.ShapeDtypeStruct((64, 64), jnp.float32),
            jax.ShapeDtypeStruct((8, 64), jnp.float32),
        ],
    )(p_aug, w1b, m1r, sc1r, be1r)


def _mlp(p_aug, w1b, m1r, sc1r, be1r, w2b, cb2r, sc2r, be2r, w3b, b3r):
    return pl.pallas_call(
        _mlp_body,
        grid=(GRID,),
        in_specs=[
            pl.BlockSpec((BLK, 8), lambda i: (i, 0)),
            pl.BlockSpec((8, 64), lambda i: (0, 0)),
            pl.BlockSpec((8, 64), lambda i: (0, 0)),
            pl.BlockSpec((8, 64), lambda i: (0, 0)),
            pl.BlockSpec((8, 64), lambda i: (0, 0)),
            pl.BlockSpec((64, 128), lambda i: (0, 0)),
            pl.BlockSpec((8, 128), lambda i: (0, 0)),
            pl.BlockSpec((8, 128), lambda i: (0, 0)),
            pl.BlockSpec((8, 128), lambda i: (0, 0)),
            pl.BlockSpec((128, 384), lambda i: (0, 0)),
            pl.BlockSpec((8, 384), lambda i: (0, 0)),
        ],
        out_specs=pl.BlockSpec((BLK // PP, 384), lambda i: (i, 0)),
        out_shape=jax.ShapeDtypeStruct((ROWS // PP, 384), jnp.float32),
    )(p_aug, w1b, m1r, sc1r, be1r, w2b, cb2r, sc2r, be2r, w3b, b3r)


def kernel(x, W1, b1, g1, be1, W2, b2, g2, be2, W3, b3):
    eps = 1e-5
    f32 = jnp.float32
    xs = x[:, :, 0]
    ys = x[:, :, 1]
    zs = x[:, :, 2]
    cx, cy, cz, px, py, pz = _sc_call(xs, ys, zs)
    center_pos = jnp.stack([cx, cy, cz], axis=-1)

    p = jnp.stack([px, py, pz], axis=-1).reshape(ROWS, 3)
    p_aug = jnp.concatenate(
        [p, jnp.ones((ROWS, 1), f32), jnp.zeros((ROWS, 4), f32)],
        axis=-1).astype(jnp.bfloat16)

    rn = float(ROWS)
    w1a = jnp.zeros((8, 64), f32)
    w1a = w1a.at[0:3, :].set(W1.T)
    w1a = w1a.at[3, :].set(b1)
    w1b = w1a.astype(jnp.bfloat16)
    w1f = w1b.astype(f32)
    m0 = _moments(p_aug)
    m1 = (m0[:, 3] / rn) @ w1f
    e2 = jnp.sum(w1f * ((m0 / rn) @ w1f), axis=0)
    v1 = e2 - m1 * m1
    sc1 = g1 / jnp.sqrt(v1 + eps)

    m1r = _row8(m1, 64)
    sc1r = _row8(sc1, 64)
    be1r = _row8(be1, 64)

    w2b = W2.T.astype(jnp.bfloat16)
    w2f = w2b.astype(f32)
    gm, sm = _stats1(p_aug, w1b, m1r, sc1r, be1r)
    s1a = sm[0] / rn
    m2 = s1a @ w2f + b2
    e2b = (jnp.sum(w2f * ((gm / rn) @ w2f), axis=0)
           + 2.0 * b2 * (s1a @ w2f) + b2 * b2)
    v2 = e2b - m2 * m2
    sc2 = g2 / jnp.sqrt(v2 + eps)

    cb2r = _row8(b2 - m2, 128)
    sc2r = _row8(sc2, 128)
    be2r = _row8(be2, 128)
    w3b = W3.T.astype(jnp.bfloat16)
    b3r = _row8(b3, 384)

    out = _mlp(p_aug, w1b, m1r, sc1r, be1r, w2b, cb2r, sc2r, be2r, w3b, b3r)
    emb = out.reshape(B, NP, 384)
    return (emb, center_pos)
